# Initial kernel scaffold; baseline (speedup 1.0000x reference)
import jax
import jax.numpy as jnp
from jax.experimental import pallas as pl


def kernel(x, cat_emb, num_emb, cat_bias, num_bias, W1, b1, W2, b2, W3, b3, W4, b4):
    raise NotImplementedError("write your pallas kernel here")


# speedup vs baseline: 2.4237x; 2.4237x over previous
import functools

import jax
import jax.numpy as jnp
from jax import lax
from jax.experimental import pallas as pl
from jax.experimental.pallas import tpu as pltpu
from jax.experimental.pallas import tpu_sc as plsc

B = 16384
NF = 17
D = 64
HDIM = NF * D
NW = 32
BPW = B // NW
NCH = 4
CH = BPW // NCH
BW = 16

def _sc_mesh():
    return plsc.VectorSubcoreMesh(core_axis_name="c", subcore_axis_name="s")


def _sc_gather_body(xr_hbm, *args):
    tabs = args[0:NF]
    btabs = args[NF:2 * NF]
    h_hbm = args[2 * NF]
    bacc_hbm = args[2 * NF + 1]
    idx_v = args[2 * NF + 2]
    rows = args[2 * NF + 3]
    bbuf = args[2 * NF + 4]
    bacc = args[2 * NF + 5]
    sem_g = args[2 * NF + 6]
    sem_w = args[2 * NF + 7]

    cid = lax.axis_index("c")
    sid = lax.axis_index("s")
    wid = sid * 2 + cid
    base = wid * BPW

    pltpu.sync_copy(xr_hbm.at[:, wid], idx_v)

    for f in range(NF):
        handles = []
        for j in range(NCH):
            handles.append(pltpu.async_copy(
                tabs[f].at[idx_v.at[f, j]], rows.at[pl.ds(j * CH, CH)], sem_g))
            handles.append(pltpu.async_copy(
                btabs[f].at[idx_v.at[f, j]], bbuf.at[pl.ds(j * CH, CH)], sem_g))
        for hd in handles:
            hd.wait()

        wh = pltpu.async_copy(
            rows, h_hbm.at[f, pl.ds(base, BPW)], sem_w)

        first = (f == 0)

        def _acc(i, carry):
            for u in range(8):
                r = i * 8 + u
                if first:
                    bacc[r, :] = bbuf[r, :]
                else:
                    bacc[r, :] = bacc[r, :] + bbuf[r, :]
            return carry

        lax.fori_loop(0, BPW // 8, _acc, 0)
        wh.wait()

    pltpu.sync_copy(bacc, bacc_hbm.at[pl.ds(base, BPW)])


def _sc_gather(xr, tabs, btabs):
    kfn = functools.partial(
        pl.kernel,
        mesh=_sc_mesh(),
        out_type=[
            jax.ShapeDtypeStruct((NF, B, D), jnp.float32),
            jax.ShapeDtypeStruct((B, BW), jnp.float32),
        ],
        scratch_types=[
            pltpu.VMEM((NF, NCH, CH), jnp.int32),
            pltpu.VMEM((BPW, D), jnp.float32),
            pltpu.VMEM((BPW, BW), jnp.float32),
            pltpu.VMEM((BPW, BW), jnp.float32),
            pltpu.SemaphoreType.DMA,
            pltpu.SemaphoreType.DMA,
        ],
        compiler_params=pltpu.CompilerParams(use_tc_tiling_on_sc=False),
    )(_sc_gather_body)
    return kfn(xr, *tabs, *btabs)


TB = 512


def _tc_body(*refs):
    h_refs = refs[0:NF]
    bacc_ref = refs[NF]
    w1, b1, w2, b2, w3, b3, w4, b4 = refs[NF + 1:NF + 9]
    out_ref = refs[NF + 9]

    feats = [r[0] for r in h_refs]
    h = jnp.concatenate(feats, axis=-1)

    s = feats[0]
    for f in range(1, NF):
        s = s + feats[f]
    fm = 0.5 * (jnp.sum(s * s, axis=1, keepdims=True)
                - jnp.sum(h * h, axis=1, keepdims=True))

    a = h
    for (w, b) in ((w1, b1), (w2, b2), (w3, b3)):
        a = jnp.dot(a, w[...], preferred_element_type=jnp.float32) + b[...]
        a = jnp.where(a >= 0, a, 0.01 * a)
    o = jnp.sum(a * w4[...], axis=1, keepdims=True) + b4[...]

    bias = jnp.sum(bacc_ref[...], axis=1, keepdims=True)
    out_ref[...] = o + bias + fm


def _tc_fused(h3, bacc, w1t, b1, w2t, b2, w3t, b3, w4, b4):
    grid = (B // TB,)
    full = lambda shape: pl.BlockSpec(shape, lambda i: (0, 0))
    return pl.pallas_call(
        _tc_body,
        grid=grid,
        in_specs=[
            pl.BlockSpec((1, TB, D), lambda i, f=f: (f, i, 0))
            for f in range(NF)
        ] + [
            pl.BlockSpec((TB, BW), lambda i: (i, 0)),
            full(w1t.shape), full(b1.shape),
            full(w2t.shape), full(b2.shape),
            full(w3t.shape), full(b3.shape),
            full(w4.shape), full(b4.shape),
        ],
        out_specs=pl.BlockSpec((TB, 1), lambda i: (i, 0)),
        out_shape=jax.ShapeDtypeStruct((B, 1), jnp.float32),
    )(*([h3] * NF), bacc, w1t, b1, w2t, b2, w3t, b3, w4, b4)


def kernel(x, cat_emb, num_emb, cat_bias, num_bias, W1, b1, W2, b2, W3, b3, W4, b4):
    tabs = list(num_emb) + [cat_emb[8 - i] for i in range(9)]
    btabs_raw = list(num_bias) + [cat_bias[8 - i] for i in range(9)]
    cols = list(range(8)) + [16 - i for i in range(9)]

    xr = x[:, jnp.array(cols)].T.reshape(NF, NW, NCH, CH)
    btabs = [jnp.pad(bt, ((0, 0), (0, BW - 1))) for bt in btabs_raw]

    h, bacc = _sc_gather(xr, tabs, btabs)

    return _tc_fused(
        h, bacc,
        W1.T, b1[None, :], W2.T, b2[None, :], W3.T, b3[None, :],
        W4, b4[None, :])


# speedup vs baseline: 2.5390x; 1.0476x over previous
import functools

import jax
import jax.numpy as jnp
from jax import lax
from jax.experimental import pallas as pl
from jax.experimental.pallas import tpu as pltpu
from jax.experimental.pallas import tpu_sc as plsc

B = 16384
NF = 17
D = 64
HDIM = NF * D
NW = 32
BPW = B // NW
NCH = 4
CH = BPW // NCH
BW = 16

def _sc_mesh():
    return plsc.VectorSubcoreMesh(core_axis_name="c", subcore_axis_name="s")


def _sc_gather_body(xr_hbm, *args):
    tabs = args[0:NF]
    btabs = args[NF:2 * NF]
    h_hbm = args[2 * NF]
    bacc_hbm = args[2 * NF + 1]
    idx_v = args[2 * NF + 2]
    rows2 = args[2 * NF + 3:2 * NF + 5]
    bbuf2 = args[2 * NF + 5:2 * NF + 7]
    bacc = args[2 * NF + 7]
    sem_g = args[2 * NF + 8]
    sem_w = args[2 * NF + 9]

    cid = lax.axis_index("c")
    sid = lax.axis_index("s")
    wid = sid * 2 + cid
    base = wid * BPW

    pltpu.sync_copy(xr_hbm.at[:, wid], idx_v)

    def fire(f):
        hs = []
        for j in range(NCH):
            hs.append(pltpu.async_copy(
                tabs[f].at[idx_v.at[f, j]],
                rows2[f % 2].at[pl.ds(j * CH, CH)], sem_g))
            hs.append(pltpu.async_copy(
                btabs[f].at[idx_v.at[f, j]],
                bbuf2[f % 2].at[pl.ds(j * CH, CH)], sem_g))
        return hs

    def acc_bias(f):
        bbuf = bbuf2[f % 2]
        first = (f == 0)

        def _acc(i, carry):
            for u in range(8):
                r = i * 8 + u
                if first:
                    bacc[r, :] = bbuf[r, :]
                else:
                    bacc[r, :] = bacc[r, :] + bbuf[r, :]
            return carry

        lax.fori_loop(0, BPW // 8, _acc, 0)

    pend = fire(0)
    prev_w = None
    for f in range(NF):
        if prev_w is not None:
            prev_w.wait()
        if f + 1 < NF:
            nxt = fire(f + 1)
        for hd in pend:
            hd.wait()
        prev_w = pltpu.async_copy(
            rows2[f % 2], h_hbm.at[f, pl.ds(base, BPW)], sem_w)
        acc_bias(f)
        pend = nxt if f + 1 < NF else []
    prev_w.wait()

    pltpu.sync_copy(bacc, bacc_hbm.at[pl.ds(base, BPW)])


def _sc_gather(xr, tabs, btabs):
    kfn = functools.partial(
        pl.kernel,
        mesh=_sc_mesh(),
        out_type=[
            jax.ShapeDtypeStruct((NF, B, D), jnp.float32),
            jax.ShapeDtypeStruct((B, BW), jnp.float32),
        ],
        scratch_types=[
            pltpu.VMEM((NF, NCH, CH), jnp.int32),
            pltpu.VMEM((BPW, D), jnp.float32),
            pltpu.VMEM((BPW, D), jnp.float32),
            pltpu.VMEM((BPW, BW), jnp.float32),
            pltpu.VMEM((BPW, BW), jnp.float32),
            pltpu.VMEM((BPW, BW), jnp.float32),
            pltpu.SemaphoreType.DMA,
            pltpu.SemaphoreType.DMA,
        ],
        compiler_params=pltpu.CompilerParams(use_tc_tiling_on_sc=False),
    )(_sc_gather_body)
    return kfn(xr, *tabs, *btabs)


TB = 512


def _tc_body(*refs):
    h_refs = refs[0:NF]
    bacc_ref = refs[NF]
    w1, b1, w2, b2, w3, b3, w4, b4 = refs[NF + 1:NF + 9]
    out_ref = refs[NF + 9]

    feats = [r[0] for r in h_refs]
    h = jnp.concatenate(feats, axis=-1)

    s = feats[0]
    for f in range(1, NF):
        s = s + feats[f]
    fm = 0.5 * (jnp.sum(s * s, axis=1, keepdims=True)
                - jnp.sum(h * h, axis=1, keepdims=True))

    a = h
    for (w, b) in ((w1, b1), (w2, b2), (w3, b3)):
        a = jnp.dot(a, w[...], preferred_element_type=jnp.float32) + b[...]
        a = jnp.where(a >= 0, a, 0.01 * a)
    o = jnp.sum(a * w4[...], axis=1, keepdims=True) + b4[...]

    bias = jnp.sum(bacc_ref[...], axis=1, keepdims=True)
    out_ref[...] = o + bias + fm


def _tc_fused(h3, bacc, w1t, b1, w2t, b2, w3t, b3, w4, b4):
    grid = (B // TB,)
    full = lambda shape: pl.BlockSpec(shape, lambda i: (0, 0))
    return pl.pallas_call(
        _tc_body,
        grid=grid,
        in_specs=[
            pl.BlockSpec((1, TB, D), lambda i, f=f: (f, i, 0))
            for f in range(NF)
        ] + [
            pl.BlockSpec((TB, BW), lambda i: (i, 0)),
            full(w1t.shape), full(b1.shape),
            full(w2t.shape), full(b2.shape),
            full(w3t.shape), full(b3.shape),
            full(w4.shape), full(b4.shape),
        ],
        out_specs=pl.BlockSpec((TB, 1), lambda i: (i, 0)),
        out_shape=jax.ShapeDtypeStruct((B, 1), jnp.float32),
    )(*([h3] * NF), bacc, w1t, b1, w2t, b2, w3t, b3, w4, b4)


def kernel(x, cat_emb, num_emb, cat_bias, num_bias, W1, b1, W2, b2, W3, b3, W4, b4):
    tabs = list(num_emb) + [cat_emb[8 - i] for i in range(9)]
    btabs_raw = list(num_bias) + [cat_bias[8 - i] for i in range(9)]
    cols = list(range(8)) + [16 - i for i in range(9)]

    xr = x[:, jnp.array(cols)].T.reshape(NF, NW, NCH, CH)
    btabs = [jnp.pad(bt, ((0, 0), (0, BW - 1))) for bt in btabs_raw]

    h, bacc = _sc_gather(xr, tabs, btabs)

    return _tc_fused(
        h, bacc,
        W1.T, b1[None, :], W2.T, b2[None, :], W3.T, b3[None, :],
        W4, b4[None, :])
